# baseline (speedup 1.0000x reference)
import jax
import jax.numpy as jnp
from jax import lax
from jax.experimental import pallas as pl
from jax.experimental.pallas import tpu as pltpu
from jax.experimental.pallas import tpu_sc as plsc

N = 4096
D = 128
K = 20

BQ = 256
NCHT = 4
JTH = 8


def _d2_kernel(b_blk_ref, b_all_ref, ncrow_ref, t3_ref, t_ref):
    q = b_blk_ref[...]
    call = b_all_ref[...]
    g = lax.dot_general(
        call, q, (((1,), (1,)), ((), ())),
        preferred_element_type=jnp.float32,
        precision=lax.Precision.HIGHEST,
    )
    d = ncrow_ref[...] - 2.0 * g
    t3_ref[...] = d[None]
    t = None
    csz = N // NCHT
    for ch in range(NCHT):
        xk = d[ch * csz:(ch + 1) * csz]
        for _ in range(JTH - 1):
            m = jnp.min(xk, axis=0, keepdims=True)
            xk = jnp.where(xk == m, jnp.inf, xk)
        mm = jnp.min(xk, axis=0, keepdims=True)
        t = mm if t is None else jnp.maximum(t, mm)
    t_ref[...] = t


CHC = 256
NCH = N // CHC
CAP = 96
NVC = CAP // 16
GPT = 8
NC_SC = 2


def _msel(m, ak, av, bk, bv):
    return jnp.where(m, ak, bk), jnp.where(m, av, bv)


def _merge2(ak, av, bk, bv):
    rbk = lax.rev(bk, (0,))
    rbv = lax.rev(bv, (0,))
    m = ak <= rbk
    lk, lv = _msel(m, ak, av, rbk, rbv)
    hk, hv = _msel(m, rbk, rbv, ak, av)
    lk, lv = plsc.sort_key_val(lk, lv)
    hk, hv = plsc.sort_key_val(hk, hv)
    return lk, lv, hk, hv


def _incorp(t0k, t0v, t1k, t1v, vk, vv):
    rvk = lax.rev(vk, (0,))
    rvv = lax.rev(vv, (0,))
    m = t1k <= rvk
    lk, lv = _msel(m, t1k, t1v, rvk, rvv)
    lk, lv = plsc.sort_key_val(lk, lv)
    return _merge2(t0k, t0v, lk, lv)


def _sc_topk_body(t3_hbm, t_hbm, out_hbm, bufa, bufb, tvm, skey_v, sidx_v,
                  obuf_v, sema, semb):
    wid = lax.axis_index("s") * NC_SC + lax.axis_index("c")
    big_g = wid // 2
    half = wid % 2
    q0 = big_g * BQ + half * 128
    lane = lax.iota(jnp.int32, 16)
    zeros16 = jnp.zeros((16,), jnp.int32)
    big = jnp.full((16,), 3.0e38, jnp.float32)

    pltpu.sync_copy(t_hbm.at[0, pl.ds(q0, 128)], tvm)

    def pf(i, c_):
        skey_v[pl.ds(i * 16, 16)] = big
        return c_
    lax.fori_loop(0, CAP * GPT, pf, 0)

    def chunk_src(ci):
        return t3_hbm.at[big_g, pl.ds(ci * CHC, CHC), pl.ds(half * 128, 128)]

    def process(buf, ws, ci):
        new_ws = []
        base_vec = jnp.full((16,), ci * CHC, jnp.int32)
        for sub in range(GPT):
            tsub = tvm[pl.ds(sub * 16, 16)]
            qlane = sub * 16 + lane

            def mkloop(tsub=tsub, qlane=qlane, sub=sub, buf=buf, w0=ws[sub]):
                @plsc.parallel_loop(0, CHC // 8, carry=w0)
                def cb(c, w):
                    guard = w < CAP
                    wu = w
                    for u in range(8):
                        col = c * 8 + u
                        v = buf[col, pl.ds(sub * 16, 16)]
                        m = (v < tsub) & guard
                        addr = wu * 128 + qlane
                        plsc.store_scatter(skey_v, [addr], v, mask=m)
                        plsc.store_scatter(sidx_v, [addr], base_vec + col,
                                           mask=m)
                        wu = wu + jnp.where(m, 1, 0)
                    return wu
                return cb
            new_ws.append(mkloop())
        return tuple(new_ws)

    pltpu.async_copy(chunk_src(0), bufa, sema)
    ws0 = (zeros16,) * GPT

    def pair(ci2, ws):
        cia = ci2 * 2
        pltpu.async_copy(chunk_src(cia + 1), bufb, semb)
        pltpu.make_async_copy(chunk_src(cia), bufa, sema).wait()
        ws = process(bufa, ws, cia)

        @pl.when(ci2 < NCH // 2 - 1)
        def _():
            pltpu.async_copy(chunk_src(cia + 2), bufa, sema)
        pltpu.make_async_copy(chunk_src(cia + 1), bufb, semb).wait()
        ws = process(bufb, ws, cia + 1)
        return ws
    lax.fori_loop(0, NCH // 2, pair, ws0)

    def sm(sub, c_):
        for l in range(16):
            ql = sub * 16 + l
            ks, vs = [], []
            for j in range(NVC):
                idxv = (j * 16 + lane) * 128 + ql
                kj = plsc.load_gather(skey_v, [idxv])
                vj = plsc.load_gather(sidx_v, [idxv])
                kj, vj = plsc.sort_key_val(kj, vj)
                ks.append(kj)
                vs.append(vj)
            t0k, t0v, t1k, t1v = _merge2(ks[0], vs[0], ks[1], vs[1])
            for j in range(2, NVC):
                t0k, t0v, t1k, t1v = _incorp(t0k, t0v, t1k, t1v, ks[j], vs[j])
            obuf_v[pl.ds(ql * 32, 16)] = t0v.astype(jnp.float32)
            obuf_v[pl.ds(ql * 32 + 16, 16)] = t1v.astype(jnp.float32)
        return c_
    lax.fori_loop(0, GPT, sm, 0)
    pltpu.sync_copy(obuf_v, out_hbm.at[pl.ds(q0 * 32, 128 * 32)])


def kernel(x, barycenters, k, batch_size):
    del x, k, batch_size
    b = barycenters
    ncrow = jnp.sum(b * b, axis=1)[:, None]
    t3, t = pl.pallas_call(
        _d2_kernel,
        grid=(N // BQ,),
        in_specs=[
            pl.BlockSpec((BQ, D), lambda i: (i, 0)),
            pl.BlockSpec((N, D), lambda i: (0, 0)),
            pl.BlockSpec((N, 1), lambda i: (0, 0)),
        ],
        out_specs=[
            pl.BlockSpec((1, N, BQ), lambda i: (i, 0, 0)),
            pl.BlockSpec((1, BQ), lambda i: (0, i)),
        ],
        out_shape=[
            jax.ShapeDtypeStruct((N // BQ, N, BQ), jnp.float32),
            jax.ShapeDtypeStruct((1, N), jnp.float32),
        ],
    )(b, b, ncrow)

    topk = pl.kernel(
        _sc_topk_body,
        out_type=jax.ShapeDtypeStruct((N * 32,), jnp.float32),
        mesh=plsc.VectorSubcoreMesh(core_axis_name="c", subcore_axis_name="s"),
        compiler_params=pltpu.CompilerParams(needs_layout_passes=False),
        scratch_types=[
            pltpu.VMEM((CHC, 128), jnp.float32),
            pltpu.VMEM((CHC, 128), jnp.float32),
            pltpu.VMEM((128,), jnp.float32),
            pltpu.VMEM(((CAP + 8) * 128,), jnp.float32),
            pltpu.VMEM(((CAP + 8) * 128,), jnp.int32),
            pltpu.VMEM((128 * 32,), jnp.float32),
            pltpu.SemaphoreType.DMA,
            pltpu.SemaphoreType.DMA,
        ],
    )(t3, t)
    return topk.reshape(N, 32)[:, :K]


# speedup vs baseline: 1.3097x; 1.3097x over previous
import jax
import jax.numpy as jnp
from jax import lax
from jax.experimental import pallas as pl
from jax.experimental.pallas import tpu as pltpu
from jax.experimental.pallas import tpu_sc as plsc

N = 4096
D = 128
K = 20

BQ = 256
NCHT = 4
JTH = 8


def _d2_kernel(b_blk_ref, b_all_ref, ncrow_ref, t3_ref, t_ref):
    q = b_blk_ref[...]
    call = b_all_ref[...]
    g = lax.dot_general(
        call, q, (((1,), (1,)), ((), ())),
        preferred_element_type=jnp.float32,
        precision=lax.Precision.HIGHEST,
    )
    d = ncrow_ref[...] - 2.0 * g
    t3_ref[...] = d[None]
    t = None
    csz = N // NCHT
    for ch in range(NCHT):
        xk = d[ch * csz:(ch + 1) * csz]
        for _ in range(JTH - 1):
            m = jnp.min(xk, axis=0, keepdims=True)
            xk = jnp.where(xk == m, jnp.inf, xk)
        mm = jnp.min(xk, axis=0, keepdims=True)
        t = mm if t is None else jnp.maximum(t, mm)
    t_ref[...] = t


def _tc_topk_kernel(t3_ref, out_ref):
    d = t3_ref[0]
    ii = lax.broadcasted_iota(jnp.int32, (N, BQ), 0)
    outs = []
    for _ in range(K):
        m = jnp.min(d, axis=0, keepdims=True)
        cand = jnp.where(d == m, ii, N)
        j = jnp.min(cand, axis=0, keepdims=True)
        outs.append(j)
        d = jnp.where(ii == j, jnp.inf, d)
    out_ref[...] = jnp.concatenate(outs, axis=0)[None].astype(jnp.float32)


CHC = 256
NCH = N // CHC
CAP = 96
NVC = CAP // 16
GPT = 4
NQSC = 2048
NC_SC = 2


def _msel(m, ak, av, bk, bv):
    return jnp.where(m, ak, bk), jnp.where(m, av, bv)


def _merge2(ak, av, bk, bv):
    rbk = lax.rev(bk, (0,))
    rbv = lax.rev(bv, (0,))
    m = ak <= rbk
    lk, lv = _msel(m, ak, av, rbk, rbv)
    hk, hv = _msel(m, rbk, rbv, ak, av)
    lk, lv = plsc.sort_key_val(lk, lv)
    hk, hv = plsc.sort_key_val(hk, hv)
    return lk, lv, hk, hv


def _incorp(t0k, t0v, t1k, t1v, vk, vv):
    rvk = lax.rev(vk, (0,))
    rvv = lax.rev(vv, (0,))
    m = t1k <= rvk
    lk, lv = _msel(m, t1k, t1v, rvk, rvv)
    lk, lv = plsc.sort_key_val(lk, lv)
    return _merge2(t0k, t0v, lk, lv)


def _sc_topk_body(t3_hbm, t_hbm, out_hbm, bufa, bufb, tvm, skey_v, sidx_v,
                  obuf_v, sema, semb):
    wid = lax.axis_index("s") * NC_SC + lax.axis_index("c")
    strip = wid // 2
    big_g = strip // 2
    hstrip = strip % 2
    h64 = wid % 2
    q0 = strip * 128 + h64 * 64
    lane = lax.iota(jnp.int32, 16)
    zeros16 = jnp.zeros((16,), jnp.int32)
    big = jnp.full((16,), 3.0e38, jnp.float32)

    pltpu.sync_copy(t_hbm.at[0, pl.ds(strip * 128, 128)], tvm)

    def pf(i, c_):
        skey_v[pl.ds(i * 16, 16)] = big
        return c_
    lax.fori_loop(0, CAP * GPT, pf, 0)

    def chunk_src(ci):
        return t3_hbm.at[big_g, pl.ds(ci * CHC, CHC), pl.ds(hstrip * 128, 128)]

    def process(buf, ws, ci):
        new_ws = []
        base_vec = jnp.full((16,), ci * CHC, jnp.int32)
        for sub in range(GPT):
            qs = h64 * 64 + sub * 16
            tsub = tvm[pl.ds(qs, 16)]
            qlane = sub * 16 + lane

            def cb(c, w, tsub=tsub, qlane=qlane, qs=qs, buf=buf):
                guard = w < CAP
                wu = w
                for u in range(8):
                    col = c * 8 + u
                    v = buf[col, pl.ds(qs, 16)]
                    m = (v < tsub) & guard
                    addr = wu * 64 + qlane
                    plsc.store_scatter(skey_v, [addr], v, mask=m)
                    plsc.store_scatter(sidx_v, [addr], base_vec + col, mask=m)
                    wu = wu + jnp.where(m, 1, 0)
                return wu
            new_ws.append(lax.fori_loop(0, CHC // 8, cb, ws[sub]))
        return tuple(new_ws)

    pltpu.async_copy(chunk_src(0), bufa, sema)
    ws0 = (zeros16,) * GPT

    def pair(ci2, ws):
        cia = ci2 * 2
        pltpu.async_copy(chunk_src(cia + 1), bufb, semb)
        pltpu.make_async_copy(chunk_src(cia), bufa, sema).wait()
        ws = process(bufa, ws, cia)

        @pl.when(ci2 < NCH // 2 - 1)
        def _():
            pltpu.async_copy(chunk_src(cia + 2), bufa, sema)
        pltpu.make_async_copy(chunk_src(cia + 1), bufb, semb).wait()
        ws = process(bufb, ws, cia + 1)
        return ws
    lax.fori_loop(0, NCH // 2, pair, ws0)

    def sm(sub, c_):
        for l in range(16):
            ql = sub * 16 + l
            ks, vs = [], []
            for j in range(NVC):
                idxv = (j * 16 + lane) * 64 + ql
                kj = plsc.load_gather(skey_v, [idxv])
                vj = plsc.load_gather(sidx_v, [idxv])
                kj, vj = plsc.sort_key_val(kj, vj)
                ks.append(kj)
                vs.append(vj)
            t0k, t0v, t1k, t1v = _merge2(ks[0], vs[0], ks[1], vs[1])
            for j in range(2, NVC):
                t0k, t0v, t1k, t1v = _incorp(t0k, t0v, t1k, t1v, ks[j], vs[j])
            obuf_v[pl.ds(ql * 32, 16)] = t0v.astype(jnp.float32)
            obuf_v[pl.ds(ql * 32 + 16, 16)] = t1v.astype(jnp.float32)
        return c_
    lax.fori_loop(0, GPT, sm, 0)
    pltpu.sync_copy(obuf_v, out_hbm.at[pl.ds(q0 * 32, 64 * 32)])


def kernel(x, barycenters, k, batch_size):
    del x, k, batch_size
    b = barycenters
    ncrow = jnp.sum(b * b, axis=1)[:, None]
    t3, t = pl.pallas_call(
        _d2_kernel,
        grid=(N // BQ,),
        in_specs=[
            pl.BlockSpec((BQ, D), lambda i: (i, 0)),
            pl.BlockSpec((N, D), lambda i: (0, 0)),
            pl.BlockSpec((N, 1), lambda i: (0, 0)),
        ],
        out_specs=[
            pl.BlockSpec((1, N, BQ), lambda i: (i, 0, 0)),
            pl.BlockSpec((1, BQ), lambda i: (0, i)),
        ],
        out_shape=[
            jax.ShapeDtypeStruct((N // BQ, N, BQ), jnp.float32),
            jax.ShapeDtypeStruct((1, N), jnp.float32),
        ],
    )(b, b, ncrow)

    sc_topk = pl.kernel(
        _sc_topk_body,
        out_type=jax.ShapeDtypeStruct((NQSC * 32,), jnp.float32),
        mesh=plsc.VectorSubcoreMesh(core_axis_name="c", subcore_axis_name="s"),
        compiler_params=pltpu.CompilerParams(needs_layout_passes=False),
        scratch_types=[
            pltpu.VMEM((CHC, 128), jnp.float32),
            pltpu.VMEM((CHC, 128), jnp.float32),
            pltpu.VMEM((128,), jnp.float32),
            pltpu.VMEM(((CAP + 8) * 64,), jnp.float32),
            pltpu.VMEM(((CAP + 8) * 64,), jnp.int32),
            pltpu.VMEM((64 * 32,), jnp.float32),
            pltpu.SemaphoreType.DMA,
            pltpu.SemaphoreType.DMA,
        ],
    )(t3, t)

    tc_topk = pl.pallas_call(
        _tc_topk_kernel,
        grid=((N - NQSC) // BQ,),
        in_specs=[pl.BlockSpec((1, N, BQ), lambda i: (i + NQSC // BQ, 0, 0))],
        out_specs=pl.BlockSpec((1, K, BQ), lambda i: (i, 0, 0)),
        out_shape=jax.ShapeDtypeStruct(((N - NQSC) // BQ, K, BQ),
                                       jnp.float32),
    )(t3)

    half1 = sc_topk.reshape(NQSC, 32)[:, :K]
    half2 = jnp.transpose(tc_topk, (0, 2, 1)).reshape(N - NQSC, K)
    return jnp.concatenate([half1, half2], axis=0)


# speedup vs baseline: 1.3860x; 1.0582x over previous
import jax
import jax.numpy as jnp
from jax import lax
from jax.experimental import pallas as pl
from jax.experimental.pallas import tpu as pltpu
from jax.experimental.pallas import tpu_sc as plsc

N = 4096
D = 128
K = 20

BQ = 256
NCHT = 4
JTH = 8


def _d2_kernel(b_blk_ref, b_all_ref, ncrow_ref, t3_ref, t_ref):
    q = b_blk_ref[...]
    call = b_all_ref[...]
    g = lax.dot_general(
        call, q, (((1,), (1,)), ((), ())),
        preferred_element_type=jnp.float32,
        precision=lax.Precision.HIGHEST,
    )
    d = ncrow_ref[...] - 2.0 * g
    t3_ref[...] = d[None]
    t = None
    csz = N // NCHT
    for ch in range(NCHT):
        xk = d[ch * csz:(ch + 1) * csz]
        for _ in range(JTH - 1):
            m = jnp.min(xk, axis=0, keepdims=True)
            xk = jnp.where(xk == m, jnp.inf, xk)
        mm = jnp.min(xk, axis=0, keepdims=True)
        t = mm if t is None else jnp.maximum(t, mm)
    t_ref[...] = t


def _d2_topk_kernel(b_blk_ref, b_all_ref, ncrow_ref, out_ref):
    q = b_blk_ref[...]
    call = b_all_ref[...]
    g = lax.dot_general(
        call, q, (((1,), (1,)), ((), ())),
        preferred_element_type=jnp.float32,
        precision=lax.Precision.HIGHEST,
    )
    d = ncrow_ref[...] - 2.0 * g
    ii = lax.broadcasted_iota(jnp.int32, (N, BQ), 0)
    outs = []
    for _ in range(K):
        m = jnp.min(d, axis=0, keepdims=True)
        cand = jnp.where(d == m, ii, N)
        j = jnp.min(cand, axis=0, keepdims=True)
        outs.append(j)
        d = jnp.where(ii == j, jnp.inf, d)
    out_ref[...] = jnp.concatenate(outs, axis=0)[None].astype(jnp.float32)


CHC = 256
NCH = N // CHC
CAP = 96
NVC = CAP // 16
GPT = 4
NQSC = 2048
NC_SC = 2


def _msel(m, ak, av, bk, bv):
    return jnp.where(m, ak, bk), jnp.where(m, av, bv)


def _merge2(ak, av, bk, bv):
    rbk = lax.rev(bk, (0,))
    rbv = lax.rev(bv, (0,))
    m = ak <= rbk
    lk, lv = _msel(m, ak, av, rbk, rbv)
    hk, hv = _msel(m, rbk, rbv, ak, av)
    lk, lv = plsc.sort_key_val(lk, lv)
    hk, hv = plsc.sort_key_val(hk, hv)
    return lk, lv, hk, hv


def _incorp(t0k, t0v, t1k, t1v, vk, vv):
    rvk = lax.rev(vk, (0,))
    rvv = lax.rev(vv, (0,))
    m = t1k <= rvk
    lk, lv = _msel(m, t1k, t1v, rvk, rvv)
    lk, lv = plsc.sort_key_val(lk, lv)
    return _merge2(t0k, t0v, lk, lv)


def _sc_topk_body(t3_hbm, t_hbm, out_hbm, bufa, bufb, tvm, skey_v, sidx_v,
                  obuf_v, sema, semb):
    wid = lax.axis_index("s") * NC_SC + lax.axis_index("c")
    strip = wid // 2
    big_g = strip // 2
    hstrip = strip % 2
    h64 = wid % 2
    q0 = strip * 128 + h64 * 64
    lane = lax.iota(jnp.int32, 16)
    zeros16 = jnp.zeros((16,), jnp.int32)
    big = jnp.full((16,), 3.0e38, jnp.float32)

    pltpu.sync_copy(t_hbm.at[0, pl.ds(strip * 128, 128)], tvm)

    def pf(i, c_):
        skey_v[pl.ds(i * 16, 16)] = big
        return c_
    lax.fori_loop(0, CAP * GPT, pf, 0)

    def chunk_src(ci):
        return t3_hbm.at[big_g, pl.ds(ci * CHC, CHC), pl.ds(hstrip * 128, 128)]

    def process(buf, ws, ci):
        new_ws = []
        base_vec = jnp.full((16,), ci * CHC, jnp.int32)
        for sub in range(GPT):
            qs = h64 * 64 + sub * 16
            tsub = tvm[pl.ds(qs, 16)]
            qlane = sub * 16 + lane

            def cb(c, w, tsub=tsub, qlane=qlane, qs=qs, buf=buf):
                guard = w < CAP
                wu = w
                for u in range(8):
                    col = c * 8 + u
                    v = buf[col, pl.ds(qs, 16)]
                    m = (v < tsub) & guard
                    addr = wu * 64 + qlane
                    plsc.store_scatter(skey_v, [addr], v, mask=m)
                    plsc.store_scatter(sidx_v, [addr], base_vec + col, mask=m)
                    wu = wu + jnp.where(m, 1, 0)
                return wu
            new_ws.append(lax.fori_loop(0, CHC // 8, cb, ws[sub]))
        return tuple(new_ws)

    pltpu.async_copy(chunk_src(0), bufa, sema)
    ws0 = (zeros16,) * GPT

    def pair(ci2, ws):
        cia = ci2 * 2
        pltpu.async_copy(chunk_src(cia + 1), bufb, semb)
        pltpu.make_async_copy(chunk_src(cia), bufa, sema).wait()
        ws = process(bufa, ws, cia)

        @pl.when(ci2 < NCH // 2 - 1)
        def _():
            pltpu.async_copy(chunk_src(cia + 2), bufa, sema)
        pltpu.make_async_copy(chunk_src(cia + 1), bufb, semb).wait()
        ws = process(bufb, ws, cia + 1)
        return ws
    lax.fori_loop(0, NCH // 2, pair, ws0)

    def sm(sub, c_):
        for l in range(16):
            ql = sub * 16 + l
            ks, vs = [], []
            for j in range(NVC):
                idxv = (j * 16 + lane) * 64 + ql
                kj = plsc.load_gather(skey_v, [idxv])
                vj = plsc.load_gather(sidx_v, [idxv])
                kj, vj = plsc.sort_key_val(kj, vj)
                ks.append(kj)
                vs.append(vj)
            t0k, t0v, t1k, t1v = _merge2(ks[0], vs[0], ks[1], vs[1])
            for j in range(2, NVC):
                t0k, t0v, t1k, t1v = _incorp(t0k, t0v, t1k, t1v, ks[j], vs[j])
            obuf_v[pl.ds(ql * 32, 16)] = t0v.astype(jnp.float32)
            obuf_v[pl.ds(ql * 32 + 16, 16)] = t1v.astype(jnp.float32)
        return c_
    lax.fori_loop(0, GPT, sm, 0)
    pltpu.sync_copy(obuf_v, out_hbm.at[pl.ds(q0 * 32, 64 * 32)])


def kernel(x, barycenters, k, batch_size):
    del x, k, batch_size
    b = barycenters
    ncrow = jnp.sum(b * b, axis=1)[:, None]
    t3, t = pl.pallas_call(
        _d2_kernel,
        grid=(NQSC // BQ,),
        in_specs=[
            pl.BlockSpec((BQ, D), lambda i: (i, 0)),
            pl.BlockSpec((N, D), lambda i: (0, 0)),
            pl.BlockSpec((N, 1), lambda i: (0, 0)),
        ],
        out_specs=[
            pl.BlockSpec((1, N, BQ), lambda i: (i, 0, 0)),
            pl.BlockSpec((1, BQ), lambda i: (0, i)),
        ],
        out_shape=[
            jax.ShapeDtypeStruct((NQSC // BQ, N, BQ), jnp.float32),
            jax.ShapeDtypeStruct((1, NQSC), jnp.float32),
        ],
    )(b, b, ncrow)

    sc_topk = pl.kernel(
        _sc_topk_body,
        out_type=jax.ShapeDtypeStruct((NQSC * 32,), jnp.float32),
        mesh=plsc.VectorSubcoreMesh(core_axis_name="c", subcore_axis_name="s"),
        compiler_params=pltpu.CompilerParams(needs_layout_passes=False),
        scratch_types=[
            pltpu.VMEM((CHC, 128), jnp.float32),
            pltpu.VMEM((CHC, 128), jnp.float32),
            pltpu.VMEM((128,), jnp.float32),
            pltpu.VMEM(((CAP + 8) * 64,), jnp.float32),
            pltpu.VMEM(((CAP + 8) * 64,), jnp.int32),
            pltpu.VMEM((64 * 32,), jnp.float32),
            pltpu.SemaphoreType.DMA,
            pltpu.SemaphoreType.DMA,
        ],
    )(t3, t)

    tc_topk = pl.pallas_call(
        _d2_topk_kernel,
        grid=((N - NQSC) // BQ,),
        in_specs=[
            pl.BlockSpec((BQ, D), lambda i: (i + NQSC // BQ, 0)),
            pl.BlockSpec((N, D), lambda i: (0, 0)),
            pl.BlockSpec((N, 1), lambda i: (0, 0)),
        ],
        out_specs=pl.BlockSpec((1, K, BQ), lambda i: (i, 0, 0)),
        out_shape=jax.ShapeDtypeStruct(((N - NQSC) // BQ, K, BQ),
                                       jnp.float32),
    )(b, b, ncrow)

    half1 = sc_topk.reshape(NQSC, 32)[:, :K]
    half2 = jnp.transpose(tc_topk, (0, 2, 1)).reshape(N - NQSC, K)
    return jnp.concatenate([half1, half2], axis=0)
